# baseline (speedup 1.0000x reference)
import functools
import numpy as np
import jax
import jax.numpy as jnp
from jax import lax
from jax.experimental import pallas as pl
from jax.experimental.pallas import tpu as pltpu
from jax.experimental.pallas import tpu_sc as plsc

K = 32
QT = 128
CH = 128
P = CH // K
N = 8192
ROWPAD = 16


def _stage(v, ix, j, k, desc=False, lex=False):
    s0 = v.shape[0]
    rest = v.shape[1:]
    g = s0 // (2 * j)
    v4 = v.reshape(g, 2, j, *rest)
    i4 = ix.reshape(g, 2, j, *rest)
    av, bv = v4[:, 0], v4[:, 1]
    ai, bi = i4[:, 0], i4[:, 1]
    gi = np.arange(g) * 2 * j
    asc = (gi & k) == 0
    if desc:
        asc = ~asc
    if lex:
        cmp = (av < bv) | ((av == bv) & (ai < bi))
    else:
        cmp = av < bv
    if asc.all():
        ceff = cmp
    elif not asc.any():
        ceff = jnp.logical_not(cmp)
    else:
        gio = lax.broadcasted_iota(jnp.int32, (g,) + (1,) * (av.ndim - 1), 0)
        flip_mask = ((gio * (2 * j)) & k) != 0
        if desc:
            flip_mask = jnp.logical_not(flip_mask)
        ceff = jnp.logical_xor(cmp, flip_mask)
    lov = jnp.where(ceff, av, bv)
    hiv = jnp.where(ceff, bv, av)
    loi = jnp.where(ceff, ai, bi)
    hii = jnp.where(ceff, bi, ai)
    v = jnp.stack([lov, hiv], axis=1).reshape(s0, *rest)
    ix = jnp.stack([loi, hii], axis=1).reshape(s0, *rest)
    return v, ix


def _sort32(v, ix, desc=False, lex=False):
    for k in (2, 4, 8, 16, 32):
        j = k // 2
        while j >= 1:
            v, ix = _stage(v, ix, j, k, desc, lex)
            j //= 2
    return v, ix


def _merge_top32(bv, bi, cv, ci, lex=False):
    if lex:
        cmp = (bv < cv) | ((bv == cv) & (bi < ci))
    else:
        cmp = bv < cv
    v = jnp.where(cmp, bv, cv)
    ix = jnp.where(cmp, bi, ci)
    for j in (16, 8, 4, 2, 1):
        v, ix = _stage(v, ix, j, 0, lex=lex)
    return v, ix


def _rev0(v):
    s0 = v.shape[0]
    return jnp.concatenate([v[i:i + 1] for i in range(s0 - 1, -1, -1)], axis=0)


def _topk_body(qt_ref, x_ref, xx_ref, out_ref, d_ref):
    qt = qt_ref[0]
    x = x_ref[0]
    xx = xx_ref[0]
    qx = jnp.dot(x, qt[0:3], preferred_element_type=jnp.float32)
    d_ref[...] = (qx + qt[3:4]) + xx

    ci0 = (lax.broadcasted_iota(jnp.int32, (K, P, QT), 0) * P
           + lax.broadcasted_iota(jnp.int32, (K, P, QT), 1))
    cv, ci = _sort32(d_ref[0:CH, :].reshape(K, P, QT), ci0)

    def body(c, carry):
        bv, bi = carry
        base = pl.multiple_of(c * CH, CH)
        cv = d_ref[pl.ds(base, CH), :].reshape(K, P, QT)
        ci = (lax.broadcasted_iota(jnp.int32, (K, P, QT), 0) * P
              + lax.broadcasted_iota(jnp.int32, (K, P, QT), 1)
              + c * CH)
        cv, ci = _sort32(cv, ci, desc=True)
        return _merge_top32(bv, bi, cv, ci)

    bv = cv
    bi = ci
    for c in range(1, N // CH):
        bv, bi = body(c, (bv, bi))

    p = P
    while p > 1:
        h = p // 2
        av, ai = bv[:, :h], bi[:, :h]
        zv = _rev0(bv[:, h:p])
        zi = _rev0(bi[:, h:p])
        cmp = (av < zv) | ((av == zv) & (ai < zi))
        bv = jnp.where(cmp, av, zv)
        bi = jnp.where(cmp, ai, zi)
        for j in (16, 8, 4, 2, 1):
            bv, bi = _stage(bv, bi, j, 0, lex=True)
        p = h

    _, bi = _sort32(bv, bi, lex=True)
    out_ref[0] = bi[:, 0, :] + pl.program_id(0) * N


def _topk_indices(qt, xn2, xx):
    b, n, _ = xn2.shape
    m = qt.shape[2]
    return pl.pallas_call(
        _topk_body,
        grid=(b, m // QT),
        in_specs=[
            pl.BlockSpec((1, 4, QT), lambda i, j: (i, 0, j)),
            pl.BlockSpec((1, n, 3), lambda i, j: (i, 0, 0)),
            pl.BlockSpec((1, n, 1), lambda i, j: (i, 0, 0)),
        ],
        out_specs=pl.BlockSpec((1, K, QT), lambda i, j: (i, 0, j)),
        out_shape=jax.ShapeDtypeStruct((b, K, m), jnp.int32),
        scratch_shapes=[pltpu.VMEM((n, QT), jnp.float32)],
        compiler_params=pltpu.CompilerParams(
            dimension_semantics=("parallel", "parallel")),
    )(qt, xn2, xx)


def _sc_gather(table, idxf):
    num_b = idxf.shape[0]
    info = plsc.get_sparse_core_info()
    nw = info.num_cores * info.num_subcores
    bpw = num_b // nw
    ch = 2048
    mesh = plsc.VectorSubcoreMesh(core_axis_name="c", subcore_axis_name="s")

    @functools.partial(
        pl.kernel,
        mesh=mesh,
        out_type=jax.ShapeDtypeStruct((num_b, ROWPAD), jnp.float32),
        compiler_params=pltpu.CompilerParams(use_tc_tiling_on_sc=False),
        scratch_types=[
            pltpu.VMEM((ch,), jnp.int32),
            pltpu.VMEM((ch, ROWPAD), jnp.float32),
            pltpu.SemaphoreType.DMA,
        ],
    )
    def gather_k(table_hbm, idx_hbm, out_hbm, idx_v, rows_v, sem):
        wid = lax.axis_index("s") * info.num_cores + lax.axis_index("c")
        base = wid * bpw
        for t in range(bpw // ch):
            o = base + t * ch
            pltpu.sync_copy(idx_hbm.at[pl.ds(o, ch)], idx_v)
            pltpu.async_copy(table_hbm.at[idx_v], rows_v, sem).wait()
            pltpu.sync_copy(rows_v, out_hbm.at[pl.ds(o, ch)])

    return gather_k(table, idxf)


@jax.jit
def kernel(xyz, new_xyz):
    b, n, _ = xyz.shape
    m = new_xyz.shape[1]
    xx = jnp.sum(xyz * xyz, axis=-1, keepdims=True)
    qq = jnp.sum(new_xyz * new_xyz, axis=-1, keepdims=True)
    q_aug = jnp.concatenate([new_xyz, qq], axis=-1)
    qt = jnp.transpose(q_aug, (0, 2, 1))
    gidx = _topk_indices(qt, -2.0 * xyz, xx)
    table = jnp.pad(xyz.reshape(b * n, 3), ((0, 0), (0, ROWPAD - 3)))
    rows = _sc_gather(table, gidx.reshape(-1))
    grouped = rows[:, :3].reshape(b, K, m, 3)
    return jnp.transpose(grouped, (0, 3, 2, 1))


# speedup vs baseline: 1.6572x; 1.6572x over previous
import functools
import numpy as np
import jax
import jax.numpy as jnp
from jax import lax
from jax.experimental import pallas as pl
from jax.experimental.pallas import tpu as pltpu
from jax.experimental.pallas import tpu_sc as plsc

K = 32
QT = 128
CH = 512
P = CH // K
N = 8192
ROWPAD = 16


def _stage(v, ix, j, k, desc=False, lex=False):
    s0 = v.shape[0]
    rest = v.shape[1:]
    g = s0 // (2 * j)
    v4 = v.reshape(g, 2, j, *rest)
    i4 = ix.reshape(g, 2, j, *rest)
    av, bv = v4[:, 0], v4[:, 1]
    ai, bi = i4[:, 0], i4[:, 1]
    gi = np.arange(g) * 2 * j
    asc = (gi & k) == 0
    if desc:
        asc = ~asc
    if lex:
        cmp = (av < bv) | ((av == bv) & (ai < bi))
    else:
        cmp = av < bv
    if asc.all():
        ceff = cmp
    elif not asc.any():
        ceff = jnp.logical_not(cmp)
    else:
        gio = lax.broadcasted_iota(jnp.int32, (g,) + (1,) * (av.ndim - 1), 0)
        flip_mask = ((gio * (2 * j)) & k) != 0
        if desc:
            flip_mask = jnp.logical_not(flip_mask)
        ceff = jnp.logical_xor(cmp, flip_mask)
    lov = jnp.where(ceff, av, bv)
    hiv = jnp.where(ceff, bv, av)
    loi = jnp.where(ceff, ai, bi)
    hii = jnp.where(ceff, bi, ai)
    v = jnp.stack([lov, hiv], axis=1).reshape(s0, *rest)
    ix = jnp.stack([loi, hii], axis=1).reshape(s0, *rest)
    return v, ix


def _sort32(v, ix, desc=False, lex=False):
    for k in (2, 4, 8, 16, 32):
        j = k // 2
        while j >= 1:
            v, ix = _stage(v, ix, j, k, desc, lex)
            j //= 2
    return v, ix


def _merge_top32(bv, bi, cv, ci, lex=False):
    if lex:
        cmp = (bv < cv) | ((bv == cv) & (bi < ci))
    else:
        cmp = bv < cv
    v = jnp.where(cmp, bv, cv)
    ix = jnp.where(cmp, bi, ci)
    for j in (16, 8, 4, 2, 1):
        v, ix = _stage(v, ix, j, 0, lex=lex)
    return v, ix


def _rev0(v):
    s0 = v.shape[0]
    return jnp.concatenate([v[i:i + 1] for i in range(s0 - 1, -1, -1)], axis=0)


def _topk_body(qt_ref, x_ref, xx_ref, out_ref, d_ref):
    qt = qt_ref[0]
    x = x_ref[0]
    xx = xx_ref[0]
    qx = jnp.dot(x, qt[0:3], preferred_element_type=jnp.float32)
    d_ref[...] = (qx + qt[3:4]) + xx

    ci0 = (lax.broadcasted_iota(jnp.int32, (K, P, QT), 0) * P
           + lax.broadcasted_iota(jnp.int32, (K, P, QT), 1))
    cv, ci = _sort32(d_ref[0:CH, :].reshape(K, P, QT), ci0)

    def body(c, carry):
        bv, bi = carry
        base = pl.multiple_of(c * CH, CH)
        cv = d_ref[pl.ds(base, CH), :].reshape(K, P, QT)
        ci = (lax.broadcasted_iota(jnp.int32, (K, P, QT), 0) * P
              + lax.broadcasted_iota(jnp.int32, (K, P, QT), 1)
              + c * CH)
        cv, ci = _sort32(cv, ci, desc=True)
        return _merge_top32(bv, bi, cv, ci)

    bv = cv
    bi = ci
    for c in range(1, N // CH):
        bv, bi = body(c, (bv, bi))

    p = P
    while p > 1:
        h = p // 2
        av, ai = bv[:, :h], bi[:, :h]
        zv = _rev0(bv[:, h:p])
        zi = _rev0(bi[:, h:p])
        cmp = (av < zv) | ((av == zv) & (ai < zi))
        bv = jnp.where(cmp, av, zv)
        bi = jnp.where(cmp, ai, zi)
        for j in (16, 8, 4, 2, 1):
            bv, bi = _stage(bv, bi, j, 0, lex=True)
        p = h

    _, bi = _sort32(bv, bi, lex=True)
    out_ref[0] = bi[:, 0, :] + pl.program_id(0) * N


def _topk_indices(qt, xn2, xx):
    b, n, _ = xn2.shape
    m = qt.shape[2]
    return pl.pallas_call(
        _topk_body,
        grid=(b, m // QT),
        in_specs=[
            pl.BlockSpec((1, 4, QT), lambda i, j: (i, 0, j)),
            pl.BlockSpec((1, n, 3), lambda i, j: (i, 0, 0)),
            pl.BlockSpec((1, n, 1), lambda i, j: (i, 0, 0)),
        ],
        out_specs=pl.BlockSpec((1, K, QT), lambda i, j: (i, 0, j)),
        out_shape=jax.ShapeDtypeStruct((b, K, m), jnp.int32),
        scratch_shapes=[pltpu.VMEM((n, QT), jnp.float32)],
        compiler_params=pltpu.CompilerParams(
            dimension_semantics=("parallel", "parallel")),
    )(qt, xn2, xx)


def _sc_gather(table, idxf):
    num_b = idxf.shape[0]
    info = plsc.get_sparse_core_info()
    nw = info.num_cores * info.num_subcores
    bpw = num_b // nw
    ch = 2048
    mesh = plsc.VectorSubcoreMesh(core_axis_name="c", subcore_axis_name="s")

    @functools.partial(
        pl.kernel,
        mesh=mesh,
        out_type=jax.ShapeDtypeStruct((num_b, ROWPAD), jnp.float32),
        compiler_params=pltpu.CompilerParams(use_tc_tiling_on_sc=False),
        scratch_types=[
            pltpu.VMEM((ch,), jnp.int32),
            pltpu.VMEM((ch, ROWPAD), jnp.float32),
            pltpu.SemaphoreType.DMA,
        ],
    )
    def gather_k(table_hbm, idx_hbm, out_hbm, idx_v, rows_v, sem):
        wid = lax.axis_index("s") * info.num_cores + lax.axis_index("c")
        base = wid * bpw
        for t in range(bpw // ch):
            o = base + t * ch
            pltpu.sync_copy(idx_hbm.at[pl.ds(o, ch)], idx_v)
            pltpu.async_copy(table_hbm.at[idx_v], rows_v, sem).wait()
            pltpu.sync_copy(rows_v, out_hbm.at[pl.ds(o, ch)])

    return gather_k(table, idxf)


@jax.jit
def kernel(xyz, new_xyz):
    b, n, _ = xyz.shape
    m = new_xyz.shape[1]
    xx = jnp.sum(xyz * xyz, axis=-1, keepdims=True)
    qq = jnp.sum(new_xyz * new_xyz, axis=-1, keepdims=True)
    q_aug = jnp.concatenate([new_xyz, qq], axis=-1)
    qt = jnp.transpose(q_aug, (0, 2, 1))
    gidx = _topk_indices(qt, -2.0 * xyz, xx)
    table = jnp.pad(xyz.reshape(b * n, 3), ((0, 0), (0, ROWPAD - 3)))
    rows = _sc_gather(table, gidx.reshape(-1))
    grouped = rows[:, :3].reshape(b, K, m, 3)
    return jnp.transpose(grouped, (0, 3, 2, 1))


# speedup vs baseline: 1.6640x; 1.0041x over previous
import functools
import numpy as np
import jax
import jax.numpy as jnp
from jax import lax
from jax.experimental import pallas as pl
from jax.experimental.pallas import tpu as pltpu
from jax.experimental.pallas import tpu_sc as plsc

K = 32
QT = 128
CH = 256
P = CH // K
N = 8192
ROWPAD = 16


def _stage(v, ix, j, k, desc=False, lex=False):
    s0 = v.shape[0]
    rest = v.shape[1:]
    g = s0 // (2 * j)
    v4 = v.reshape(g, 2, j, *rest)
    i4 = ix.reshape(g, 2, j, *rest)
    av, bv = v4[:, 0], v4[:, 1]
    ai, bi = i4[:, 0], i4[:, 1]
    gi = np.arange(g) * 2 * j
    asc = (gi & k) == 0
    if desc:
        asc = ~asc
    if lex:
        cmp = (av < bv) | ((av == bv) & (ai < bi))
    else:
        cmp = av < bv
    if asc.all():
        ceff = cmp
    elif not asc.any():
        ceff = jnp.logical_not(cmp)
    else:
        gio = lax.broadcasted_iota(jnp.int32, (g,) + (1,) * (av.ndim - 1), 0)
        flip_mask = ((gio * (2 * j)) & k) != 0
        if desc:
            flip_mask = jnp.logical_not(flip_mask)
        ceff = jnp.logical_xor(cmp, flip_mask)
    lov = jnp.where(ceff, av, bv)
    hiv = jnp.where(ceff, bv, av)
    loi = jnp.where(ceff, ai, bi)
    hii = jnp.where(ceff, bi, ai)
    v = jnp.stack([lov, hiv], axis=1).reshape(s0, *rest)
    ix = jnp.stack([loi, hii], axis=1).reshape(s0, *rest)
    return v, ix


def _sort32(v, ix, desc=False, lex=False):
    for k in (2, 4, 8, 16, 32):
        j = k // 2
        while j >= 1:
            v, ix = _stage(v, ix, j, k, desc, lex)
            j //= 2
    return v, ix


def _merge_top32(bv, bi, cv, ci, lex=False):
    if lex:
        cmp = (bv < cv) | ((bv == cv) & (bi < ci))
    else:
        cmp = bv < cv
    v = jnp.where(cmp, bv, cv)
    ix = jnp.where(cmp, bi, ci)
    for j in (16, 8, 4, 2, 1):
        v, ix = _stage(v, ix, j, 0, lex=lex)
    return v, ix


def _rev0(v):
    s0 = v.shape[0]
    return jnp.concatenate([v[i:i + 1] for i in range(s0 - 1, -1, -1)], axis=0)


def _topk_body(qt_ref, x_ref, xx_ref, out_ref, d_ref):
    qt = qt_ref[0]
    x = x_ref[0]
    xx = xx_ref[0]
    qx = jnp.dot(x, qt[0:3], preferred_element_type=jnp.float32)
    d_ref[...] = (qx + qt[3:4]) + xx

    ci0 = (lax.broadcasted_iota(jnp.int32, (K, P, QT), 0) * P
           + lax.broadcasted_iota(jnp.int32, (K, P, QT), 1))
    cv, ci = _sort32(d_ref[0:CH, :].reshape(K, P, QT), ci0)

    def body(c, carry):
        bv, bi = carry
        base = pl.multiple_of(c * CH, CH)
        cv = d_ref[pl.ds(base, CH), :].reshape(K, P, QT)
        ci = (lax.broadcasted_iota(jnp.int32, (K, P, QT), 0) * P
              + lax.broadcasted_iota(jnp.int32, (K, P, QT), 1)
              + c * CH)
        cv, ci = _sort32(cv, ci, desc=True)
        return _merge_top32(bv, bi, cv, ci)

    bv = cv
    bi = ci
    for c in range(1, N // CH):
        bv, bi = body(c, (bv, bi))

    p = P
    while p > 1:
        h = p // 2
        av, ai = bv[:, :h], bi[:, :h]
        zv = _rev0(bv[:, h:p])
        zi = _rev0(bi[:, h:p])
        cmp = (av < zv) | ((av == zv) & (ai < zi))
        bv = jnp.where(cmp, av, zv)
        bi = jnp.where(cmp, ai, zi)
        for j in (16, 8, 4, 2, 1):
            bv, bi = _stage(bv, bi, j, 0, lex=True)
        p = h

    _, bi = _sort32(bv, bi, lex=True)
    out_ref[0] = bi[:, 0, :] + pl.program_id(0) * N


def _topk_indices(qt, xn2, xx):
    b, n, _ = xn2.shape
    m = qt.shape[2]
    return pl.pallas_call(
        _topk_body,
        grid=(b, m // QT),
        in_specs=[
            pl.BlockSpec((1, 4, QT), lambda i, j: (i, 0, j)),
            pl.BlockSpec((1, n, 3), lambda i, j: (i, 0, 0)),
            pl.BlockSpec((1, n, 1), lambda i, j: (i, 0, 0)),
        ],
        out_specs=pl.BlockSpec((1, K, QT), lambda i, j: (i, 0, j)),
        out_shape=jax.ShapeDtypeStruct((b, K, m), jnp.int32),
        scratch_shapes=[pltpu.VMEM((n, QT), jnp.float32)],
        compiler_params=pltpu.CompilerParams(
            dimension_semantics=("parallel", "parallel")),
    )(qt, xn2, xx)


def _sc_gather(table, idxf):
    num_b = idxf.shape[0]
    info = plsc.get_sparse_core_info()
    nw = info.num_cores * info.num_subcores
    bpw = num_b // nw
    ch = 2048
    mesh = plsc.VectorSubcoreMesh(core_axis_name="c", subcore_axis_name="s")

    @functools.partial(
        pl.kernel,
        mesh=mesh,
        out_type=jax.ShapeDtypeStruct((num_b, ROWPAD), jnp.float32),
        compiler_params=pltpu.CompilerParams(use_tc_tiling_on_sc=False),
        scratch_types=[
            pltpu.VMEM((ch,), jnp.int32),
            pltpu.VMEM((ch, ROWPAD), jnp.float32),
            pltpu.SemaphoreType.DMA,
        ],
    )
    def gather_k(table_hbm, idx_hbm, out_hbm, idx_v, rows_v, sem):
        wid = lax.axis_index("s") * info.num_cores + lax.axis_index("c")
        base = wid * bpw
        for t in range(bpw // ch):
            o = base + t * ch
            pltpu.sync_copy(idx_hbm.at[pl.ds(o, ch)], idx_v)
            pltpu.async_copy(table_hbm.at[idx_v], rows_v, sem).wait()
            pltpu.sync_copy(rows_v, out_hbm.at[pl.ds(o, ch)])

    return gather_k(table, idxf)


@jax.jit
def kernel(xyz, new_xyz):
    b, n, _ = xyz.shape
    m = new_xyz.shape[1]
    xx = jnp.sum(xyz * xyz, axis=-1, keepdims=True)
    qq = jnp.sum(new_xyz * new_xyz, axis=-1, keepdims=True)
    q_aug = jnp.concatenate([new_xyz, qq], axis=-1)
    qt = jnp.transpose(q_aug, (0, 2, 1))
    gidx = _topk_indices(qt, -2.0 * xyz, xx)
    table = jnp.pad(xyz.reshape(b * n, 3), ((0, 0), (0, ROWPAD - 3)))
    rows = _sc_gather(table, gidx.reshape(-1))
    grouped = rows[:, :3].reshape(b, K, m, 3)
    return jnp.transpose(grouped, (0, 3, 2, 1))
